# Initial kernel scaffold; baseline (speedup 1.0000x reference)
import jax
import jax.numpy as jnp
from jax.experimental import pallas as pl


def kernel(length, table):
    raise NotImplementedError("write your pallas kernel here")


# speedup vs baseline: 2.5078x; 2.5078x over previous
import functools

import jax
import jax.numpy as jnp
from jax import lax
from jax.experimental import pallas as pl
from jax.experimental.pallas import tpu as pltpu
from jax.experimental.pallas import tpu_sc as plsc

D = 256
T = 512
TROWS = 513
P_ROWS = 1024
NC = 2
NS = 16
NW = NC * NS
ROWS_PER_W = T // NW
P_PER_TILE = P_ROWS // NS

_mesh = plsc.VectorSubcoreMesh(core_axis_name="c", subcore_axis_name="s")


@functools.partial(
    pl.kernel,
    out_type=jax.ShapeDtypeStruct((T * T * D,), jnp.float32),
    mesh=_mesh,
    scratch_types=[
        pltpu.VMEM_SHARED((P_ROWS * D,), jnp.float32),
        pltpu.SemaphoreType.DMA,
    ],
)
def _rpe_sc(table_hbm, out_hbm, p_sh, sem):
    c = lax.axis_index("c")
    s = lax.axis_index("s")
    wid = s * NC + c

    p_base = s * P_PER_TILE
    fills = []
    for m in range(P_PER_TILE):
        p_row = p_base + m
        src_row = jnp.clip(p_row - 256, 0, TROWS - 1)
        fills.append(pltpu.async_copy(
            table_hbm.at[pl.ds(pl.multiple_of(src_row * D, D), D)],
            p_sh.at[pl.ds(pl.multiple_of(p_row * D, D), D)],
            sem))
    for cp in fills:
        cp.wait()
    plsc.subcore_barrier()

    i0 = wid * ROWS_PER_W
    outs = []
    for r in range(ROWS_PER_W):
        i = i0 + r
        outs.append(pltpu.async_copy(
            p_sh.at[pl.ds(pl.multiple_of((T - i) * D, D), T * D)],
            out_hbm.at[pl.ds(pl.multiple_of(i * (T * D), D), T * D)],
            sem))
    for cp in outs:
        cp.wait()


def kernel(length, table):
    del length
    out = _rpe_sc(jnp.reshape(table, (TROWS * D,)))
    return jnp.reshape(out, (T, T, D))
